# baseline (speedup 1.0000x reference)
import jax
import jax.numpy as jnp
import numpy as np
from jax.experimental import pallas as pl

_B, _N, _T, _C, _H = 4, 256, 12, 64, 4
_D = _C // _H


def _attn_body(x_ref, adjt_ref, wt_ref, a_ref, cw_ref, cb_ref, o_ref):
    adjT = adjt_ref[...]
    ones_row = jnp.ones((1, _N), dtype=jnp.float32)
    xb = x_ref[0]
    hTs = [jax.lax.dot_general(wt_ref[...], xb[:, t * _C:(t + 1) * _C],
                               (((1,), (1,)), ((), ())),
                               preferred_element_type=jnp.float32)
           for t in range(_T)]
    hT_all = jnp.concatenate(hTs, axis=1)
    edr_all = jnp.dot(a_ref[...], hT_all, preferred_element_type=jnp.float32)
    edc_all = edr_all.T

    for t in range(_T):
        sl = slice(t * _N, (t + 1) * _N)
        hT = hT_all[:, sl]
        edr = edr_all[:, sl]
        edc = edc_all[sl, :]
        dmax = jnp.max(edr[_H:2 * _H, :], axis=1, keepdims=True)
        tm = edr[:_H, :] + dmax
        m_rows = jnp.maximum(tm, 0.2 * tm)

        outs = []
        denoms = []
        for hh in range(_H):
            tt = edc[:, _H + hh][:, None] + edr[hh][None, :]
            u = jnp.maximum(tt, 0.2 * tt)
            pT = jnp.exp2(u - m_rows[hh][None, :]) * adjT
            hs = jnp.concatenate([hT[hh * _D:(hh + 1) * _D, :], ones_row],
                                 axis=0)
            agg = jnp.dot(hs, pT, preferred_element_type=jnp.float32)
            outs.append(agg[:_D, :])
            denoms.append(agg[_D:, :])
        recips = [1.0 / d for d in denoms]
        outT = jnp.concatenate([o * r for o, r in zip(outs, recips)], axis=0)
        y_t = jax.lax.dot_general(outT, cw_ref[...],
                                  (((0,), (1,)), ((), ())),
                                  preferred_element_type=jnp.float32)
        y_t = jnp.maximum(y_t + cb_ref[...], 0.0)
        o_ref[0, :, t * _C:(t + 1) * _C] = y_t


def kernel(x, adj, W, a_src, a_dst, conv_w, conv_b):
    xr = x.reshape(_B, _N, _T * _C)
    adjt = adj.T

    log2e = jnp.float32(np.log2(np.e))
    eye_h = jnp.eye(_H, dtype=jnp.float32)
    blk_src = jnp.einsum('hd,hg->ghd', a_src, eye_h).reshape(_H, _C)
    blk_dst = jnp.einsum('hd,hg->ghd', a_dst, eye_h).reshape(_H, _C)
    A = jnp.concatenate([blk_src, blk_dst], axis=0) * log2e

    cb_row = conv_b.reshape(1, _C)

    y = pl.pallas_call(
        _attn_body,
        grid=(_B,),
        in_specs=[
            pl.BlockSpec((1, _N, _T * _C), lambda i: (i, 0, 0)),
            pl.BlockSpec((_N, _N), lambda i: (0, 0)),
            pl.BlockSpec((_C, _C), lambda i: (0, 0)),
            pl.BlockSpec((2 * _H, _C), lambda i: (0, 0)),
            pl.BlockSpec((_C, _C), lambda i: (0, 0)),
            pl.BlockSpec((1, _C), lambda i: (0, 0)),
        ],
        out_specs=pl.BlockSpec((1, _N, _T * _C), lambda i: (i, 0, 0)),
        out_shape=jax.ShapeDtypeStruct((_B, _N, _T * _C), jnp.float32),
    )(xr, adjt, W.T, A, conv_w, cb_row)

    return y.reshape(_B, _N, _T, _C)


# speedup vs baseline: 1.1781x; 1.1781x over previous
import jax
import jax.numpy as jnp
import numpy as np
from jax.experimental import pallas as pl

_B, _N, _T, _C, _H = 4, 256, 12, 64, 4
_D = _C // _H
_S = 12


def _attn_body(x_ref, adjt_ref, wt_ref, a_ref, cw_ref, cb_ref, o_ref):
    adjT = adjt_ref[...]
    ones_row = jnp.ones((1, _N), dtype=jnp.float32)
    xall = x_ref[0]
    hT_all = jnp.dot(wt_ref[...], xall, preferred_element_type=jnp.float32)
    edr_all = jnp.dot(a_ref[...], hT_all, preferred_element_type=jnp.float32)
    edc_all = edr_all.T
    out_slabs = []
    for s in range(_S):
        sl = slice(s * _N, (s + 1) * _N)
        hT = hT_all[:, sl]
        edr = edr_all[:, sl]
        edc = edc_all[sl, :]
        dmax = jnp.max(edr[_H:2 * _H, :], axis=1, keepdims=True)
        tm = edr[:_H, :] + dmax
        m_rows = jnp.maximum(tm, 0.2 * tm)

        outs = []
        denoms = []
        for hh in range(_H):
            t = edc[:, _H + hh][:, None] + edr[hh][None, :]
            u = jnp.maximum(t, 0.2 * t)
            pT = jnp.exp2(u - m_rows[hh][None, :]) * adjT
            hs = jnp.concatenate([hT[hh * _D:(hh + 1) * _D, :], ones_row],
                                 axis=0)
            agg = jnp.dot(hs, pT, preferred_element_type=jnp.float32)
            outs.append(agg[:_D, :])
            denoms.append(agg[_D:, :])
        recips = [1.0 / d for d in denoms]
        out_slabs.append(
            jnp.concatenate([o * r for o, r in zip(outs, recips)], axis=0))
    outT_all = jnp.concatenate(out_slabs, axis=1)
    yT = jnp.dot(cw_ref[...], outT_all, preferred_element_type=jnp.float32)
    yT = yT + cb_ref[...]
    o_ref[0] = jnp.maximum(yT, 0.0)


def kernel(x, adj, W, a_src, a_dst, conv_w, conv_b):
    xt = (jnp.transpose(x, (0, 2, 3, 1))
          .reshape(_B * _T // _S, _S, _C, _N)
          .transpose(0, 2, 1, 3)
          .reshape(_B * _T // _S, _C, _S * _N))
    adjt = adj.T

    log2e = jnp.float32(np.log2(np.e))
    eye_h = jnp.eye(_H, dtype=jnp.float32)
    blk_src = jnp.einsum('hd,hg->ghd', a_src, eye_h).reshape(_H, _C)
    blk_dst = jnp.einsum('hd,hg->ghd', a_dst, eye_h).reshape(_H, _C)
    A = jnp.concatenate([blk_src, blk_dst], axis=0) * log2e

    cb_col = conv_b.reshape(_C, 1)

    y = pl.pallas_call(
        _attn_body,
        grid=(_B * _T // _S,),
        in_specs=[
            pl.BlockSpec((1, _C, _S * _N), lambda i: (i, 0, 0)),
            pl.BlockSpec((_N, _N), lambda i: (0, 0)),
            pl.BlockSpec((_C, _C), lambda i: (0, 0)),
            pl.BlockSpec((2 * _H, _C), lambda i: (0, 0)),
            pl.BlockSpec((_C, _C), lambda i: (0, 0)),
            pl.BlockSpec((_C, 1), lambda i: (0, 0)),
        ],
        out_specs=pl.BlockSpec((1, _C, _S * _N), lambda i: (i, 0, 0)),
        out_shape=jax.ShapeDtypeStruct((_B * _T // _S, _C, _S * _N), jnp.float32),
    )(xt, adjt, W.T, A, conv_w, cb_col)

    yr = (y.reshape(_B * _T // _S, _C, _S, _N)
          .transpose(0, 2, 1, 3)
          .reshape(_B, _T, _C, _N))
    return jnp.transpose(yr, (0, 3, 1, 2))
